# Initial kernel scaffold; baseline (speedup 1.0000x reference)
import jax
import jax.numpy as jnp
from jax.experimental import pallas as pl


def kernel(pred_dict, label_dict):
    raise NotImplementedError("write your pallas kernel here")


# speedup vs baseline: 3.5702x; 3.5702x over previous
import functools

import jax
import jax.numpy as jnp
from jax import lax
from jax.experimental import pallas as pl
from jax.experimental.pallas import tpu as pltpu
from jax.experimental.pallas import tpu_sc as plsc

N = 32768
NW = 16
CH = N // NW
SLICES = CH // 16

_F32 = jnp.float32
_I32 = jnp.int32
_U32 = jnp.uint32

_LN2 = 0.6931471805599453
_SQRT2 = 1.4142135
_MINNORM = 1.1754944e-38


def _iota16():
    return lax.iota(_I32, 16)


def _field(acc, f):
    return jnp.sum(jnp.where(_iota16() == f, acc, _F32(0.0)))


def _recip(x):
    bits = lax.bitcast_convert_type(x, _I32)
    r = lax.bitcast_convert_type(jnp.int32(0x7EF311C3) - bits, _F32)
    for _ in range(3):
        r = r * (_F32(2.0) - x * r)
    return r


def _safelog(x):
    bits = lax.bitcast_convert_type(x, _I32)
    e = ((bits >> 23) & 0xFF) - 127
    m = lax.bitcast_convert_type((bits & 0x007FFFFF) | 0x3F800000, _F32)
    big = m >= _F32(_SQRT2)
    m = jnp.where(big, m * _F32(0.5), m)
    e = jnp.where(big, e + 1, e)
    z = (m - _F32(1.0)) * _recip(m + _F32(1.0))
    z2 = z * z
    p = z * (_F32(2.0) + z2 * (_F32(2.0 / 3.0)
                               + z2 * (_F32(2.0 / 5.0) + z2 * _F32(2.0 / 7.0))))
    r = e.astype(_F32) * _F32(_LN2) + p
    return jnp.where(x < _F32(_MINNORM), _F32(-100.0), r)


def _body(pred_hbm, key_hbm, y_hbm, out_hbm,
          pred_v, key0_v, key1_v, lab_v, xbuf_w, xbuf_r, out_v, shared):
    wid = lax.axis_index("s")
    base = wid * CH
    it16 = _iota16()

    def exchange(par, vals16):
        xbuf_w[...] = vals16
        pltpu.sync_copy(xbuf_w, shared.at[par, pl.ds(wid * 16, 16)])
        plsc.subcore_barrier()
        pltpu.sync_copy(shared.at[par], xbuf_r)
        acc = jnp.zeros((16,), _F32)
        for j in range(NW):
            acc = acc + xbuf_r[pl.ds(j * 16, 16)]
        return acc

    def lanes2(a, b):
        return jnp.where(it16 == 0, a, jnp.where(it16 == 1, b, _F32(0.0)))

    pltpu.sync_copy(pred_hbm.at[pl.ds(base, CH)], pred_v)
    pltpu.sync_copy(key_hbm.at[pl.ds(base, CH)], lab_v)

    def _sum_lab(j, acc):
        for u in range(8):
            acc = acc + lab_v[pl.ds((j * 8 + u) * 16, 16)]
        return acc

    s_key = jnp.sum(lax.fori_loop(0, SLICES // 8, _sum_lab,
                                  jnp.zeros((16,), _I32)).astype(_F32))
    pltpu.sync_copy(y_hbm.at[pl.ds(base, CH)], lab_v)
    s_y = jnp.sum(lax.fori_loop(0, SLICES // 8, _sum_lab,
                                jnp.zeros((16,), _I32)).astype(_F32))

    acc = exchange(0, lanes2(s_key, s_y))
    b_i = N - _field(acc, 0).astype(_I32)
    c_i = N - _field(acc, 1).astype(_I32)

    def _mkkeys(j, carry):
        for u in range(4):
            off = (j * 4 + u) * 16
            bits = lax.bitcast_convert_type(pred_v[pl.ds(off, 16)], _I32)
            keyu = lax.bitcast_convert_type(
                jnp.where(bits < 0, jnp.invert(bits),
                          bits | jnp.int32(-2147483648)), _U32)
            gidx = base + off + it16
            key0_v[pl.ds(off, 16)] = jnp.where(gidx < b_i, keyu, _U32(0))
            key1_v[pl.ds(off, 16)] = jnp.where(gidx >= b_i, keyu, _U32(0))
        return carry

    lax.fori_loop(0, SLICES // 4, _mkkeys, 0)

    n0 = b_i
    n1 = N - b_i
    k0f = jnp.maximum(1, n0 >> 3).astype(_F32)
    k1f = jnp.maximum(1, n1 >> 3).astype(_F32)

    def _count2(t0, t1):
        def body(j, carry):
            a0, a1 = carry
            for u in range(4):
                off = (j * 4 + u) * 16
                a0 = a0 + jnp.where(key0_v[pl.ds(off, 16)] > t0,
                                    _U32(1), _U32(0))
                a1 = a1 + jnp.where(key1_v[pl.ds(off, 16)] > t1,
                                    _U32(1), _U32(0))
            return (a0, a1)
        z = jnp.zeros((16,), _U32)
        a0, a1 = lax.fori_loop(0, SLICES // 4, body, (z, z))
        return (jnp.sum(lax.bitcast_convert_type(a0, _I32).astype(_F32)),
                jnp.sum(lax.bitcast_convert_type(a1, _I32).astype(_F32)))

    def _bstep(i, carry):
        lo0, hi0, lo1, hi1 = carry
        par = (i + 1) & 1
        mid0 = lo0 + lax.shift_right_logical(hi0 - lo0, _U32(1))
        mid1 = lo1 + lax.shift_right_logical(hi1 - lo1, _U32(1))
        c0, c1 = _count2(mid0, mid1)
        accv = exchange(par, lanes2(c0, c1))
        c0g = _field(accv, 0)
        c1g = _field(accv, 1)
        up0 = c0g >= k0f
        up1 = c1g >= k1f
        lo0 = jnp.where(up0, mid0 + _U32(1), lo0)
        hi0 = jnp.where(up0, hi0, mid0)
        lo1 = jnp.where(up1, mid1 + _U32(1), lo1)
        hi1 = jnp.where(up1, hi1, mid1)
        return (lo0, hi0, lo1, hi1)

    lo0, hi0, lo1, hi1 = lax.fori_loop(
        0, 32, _bstep,
        (_U32(0), _U32(0xFFFFFFFF), _U32(0), _U32(0xFFFFFFFF)))
    t0, t1 = lo0, lo1

    def _csum(j, carry):
        a0, a1, s0, s1 = carry
        for u in range(4):
            off = (j * 4 + u) * 16
            pv = pred_v[pl.ds(off, 16)]
            m0 = key0_v[pl.ds(off, 16)] > t0
            m1 = key1_v[pl.ds(off, 16)] > t1
            a0 = a0 + jnp.where(m0, _U32(1), _U32(0))
            a1 = a1 + jnp.where(m1, _U32(1), _U32(0))
            s0 = s0 + jnp.where(m0, pv, _F32(0.0))
            s1 = s1 + jnp.where(m1, pv, _F32(0.0))
        return (a0, a1, s0, s1)

    zu = jnp.zeros((16,), _U32)
    zf = jnp.zeros((16,), _F32)
    a0, a1, s0, s1 = lax.fori_loop(0, SLICES // 4, _csum, (zu, zu, zf, zf))
    cg0 = jnp.sum(lax.bitcast_convert_type(a0, _I32).astype(_F32))
    cg1 = jnp.sum(lax.bitcast_convert_type(a1, _I32).astype(_F32))
    sg0 = jnp.sum(s0)
    sg1 = jnp.sum(s1)

    vals = (jnp.where(it16 == 0, cg0,
            jnp.where(it16 == 1, sg0,
            jnp.where(it16 == 2, cg1,
            jnp.where(it16 == 3, sg1, _F32(0.0))))))
    accf = exchange(1, vals)

    @pl.when(wid == 0)
    def _epilogue():
        C0 = _field(accf, 0)
        S0 = _field(accf, 1)
        C1 = _field(accf, 2)
        S1 = _field(accf, 3)

        tv = jnp.where(it16 == 0, jnp.full((16,), t0), jnp.full((16,), t1))
        topbit = lax.shift_right_logical(tv, _U32(31)) > _U32(0)
        tbits = jnp.where(topbit, tv & _U32(0x7FFFFFFF), ~tv)
        tval = lax.bitcast_convert_type(tbits, _F32)

        kf = jnp.where(it16 == 0, jnp.full((16,), k0f), jnp.full((16,), k1f))
        Cf = jnp.where(it16 == 0, jnp.full((16,), C0), jnp.full((16,), C1))
        Sf = jnp.where(it16 == 0, jnp.full((16,), S0), jnp.full((16,), S1))
        P = (Sf + (kf - Cf) * tval) * _recip(kf)

        ones0 = (n0 - jnp.minimum(n0, c_i)).astype(_F32)
        ones1 = (N - c_i).astype(_F32) - ones0
        nf = jnp.where(it16 == 0, jnp.full((16,), n0.astype(_F32)),
                       jnp.full((16,), n1.astype(_F32)))
        T = (jnp.where(it16 == 0, jnp.full((16,), ones0),
                       jnp.full((16,), ones1))
             * _recip(jnp.maximum(nf, _F32(1.0))))

        lp = jnp.maximum(_safelog(P), _F32(-100.0))
        l1p = jnp.maximum(_safelog(_F32(1.0) - P), _F32(-100.0))
        term = -(T * lp + (_F32(1.0) - T) * l1p)

        maskv = (it16 < 2) & (nf > _F32(0.5))
        nseg = jnp.sum(jnp.where(maskv, _F32(1.0), _F32(0.0)))
        loss = jnp.sum(jnp.where(maskv, term, _F32(0.0))) * _recip(nseg)
        out_v[...] = jnp.full((16,), loss)
        pltpu.sync_copy(out_v, out_hbm)


_mil = functools.partial(
    pl.kernel,
    out_type=jax.ShapeDtypeStruct((16,), _F32),
    mesh=plsc.VectorSubcoreMesh(core_axis_name="c", subcore_axis_name="s",
                                num_cores=1),
    compiler_params=pltpu.CompilerParams(needs_layout_passes=False),
    scratch_types=[
        pltpu.VMEM((CH,), _F32),
        pltpu.VMEM((CH,), _U32),
        pltpu.VMEM((CH,), _U32),
        pltpu.VMEM((CH,), _I32),
        pltpu.VMEM((16,), _F32),
        pltpu.VMEM((NW * 16,), _F32),
        pltpu.VMEM((16,), _F32),
        pltpu.VMEM_SHARED((2, NW * 16), _F32),
    ],
)(_body)


def kernel(pred_dict, label_dict):
    pred = pred_dict[0, :, 0]
    y_row = label_dict[0].astype(_I32)
    seg_key = label_dict[1].astype(_I32)
    out = _mil(pred, seg_key, y_row)
    return out[0]


# speedup vs baseline: 3.6478x; 1.0217x over previous
import functools

import jax
import jax.numpy as jnp
from jax import lax
from jax.experimental import pallas as pl
from jax.experimental.pallas import tpu as pltpu
from jax.experimental.pallas import tpu_sc as plsc

N = 32768
NW = 16
CH = N // NW
SLICES = CH // 16

_F32 = jnp.float32
_I32 = jnp.int32
_U32 = jnp.uint32

_LN2 = 0.6931471805599453
_SQRT2 = 1.4142135
_MINNORM = 1.1754944e-38


def _iota16():
    return lax.iota(_I32, 16)


def _field(acc, f):
    return jnp.sum(jnp.where(_iota16() == f, acc, _F32(0.0)))


def _recip(x):
    bits = lax.bitcast_convert_type(x, _I32)
    r = lax.bitcast_convert_type(jnp.int32(0x7EF311C3) - bits, _F32)
    for _ in range(3):
        r = r * (_F32(2.0) - x * r)
    return r


def _safelog(x):
    bits = lax.bitcast_convert_type(x, _I32)
    e = ((bits >> 23) & 0xFF) - 127
    m = lax.bitcast_convert_type((bits & 0x007FFFFF) | 0x3F800000, _F32)
    big = m >= _F32(_SQRT2)
    m = jnp.where(big, m * _F32(0.5), m)
    e = jnp.where(big, e + 1, e)
    z = (m - _F32(1.0)) * _recip(m + _F32(1.0))
    z2 = z * z
    p = z * (_F32(2.0) + z2 * (_F32(2.0 / 3.0)
                               + z2 * (_F32(2.0 / 5.0) + z2 * _F32(2.0 / 7.0))))
    r = e.astype(_F32) * _F32(_LN2) + p
    return jnp.where(x < _F32(_MINNORM), _F32(-100.0), r)


def _body(pred_hbm, key_hbm, y_hbm, out_hbm,
          pred_v, key0_v, key1_v, lab_v, xbuf_w, xbuf_r, out_v, shared):
    wid = lax.axis_index("s")
    base = wid * CH
    it16 = _iota16()

    def exchange(par, vals16):
        xbuf_w[...] = vals16
        pltpu.sync_copy(xbuf_w, shared.at[par, pl.ds(wid * 16, 16)])
        plsc.subcore_barrier()
        pltpu.sync_copy(shared.at[par], xbuf_r)
        parts = [xbuf_r[pl.ds(j * 16, 16)] for j in range(NW)]
        while len(parts) > 1:
            parts = [parts[i] + parts[i + 1] for i in range(0, len(parts), 2)]
        return parts[0]

    def lanes2(a, b):
        return jnp.where(it16 == 0, a, jnp.where(it16 == 1, b, _F32(0.0)))

    pltpu.sync_copy(pred_hbm.at[pl.ds(base, CH)], pred_v)
    pltpu.sync_copy(key_hbm.at[pl.ds(base, CH)], lab_v)

    def _sum_lab(j, accs):
        return tuple(accs[u] + lab_v[pl.ds((j * 8 + u) * 16, 16)]
                     for u in range(8))

    def _tree(parts):
        parts = list(parts)
        while len(parts) > 1:
            parts = [parts[i] + parts[i + 1] for i in range(0, len(parts), 2)]
        return parts[0]

    zi8 = (jnp.zeros((16,), _I32),) * 8
    s_key = jnp.sum(_tree(lax.fori_loop(0, SLICES // 8, _sum_lab,
                                        zi8)).astype(_F32))
    pltpu.sync_copy(y_hbm.at[pl.ds(base, CH)], lab_v)
    s_y = jnp.sum(_tree(lax.fori_loop(0, SLICES // 8, _sum_lab,
                                      zi8)).astype(_F32))

    acc = exchange(0, lanes2(s_key, s_y))
    b_i = N - _field(acc, 0).astype(_I32)
    c_i = N - _field(acc, 1).astype(_I32)

    def _mkkeys(j, carry):
        for u in range(4):
            off = (j * 4 + u) * 16
            bits = lax.bitcast_convert_type(pred_v[pl.ds(off, 16)], _I32)
            keyu = lax.bitcast_convert_type(
                jnp.where(bits < 0, jnp.invert(bits),
                          bits | jnp.int32(-2147483648)), _U32)
            gidx = base + off + it16
            key0_v[pl.ds(off, 16)] = jnp.where(gidx < b_i, keyu, _U32(0))
            key1_v[pl.ds(off, 16)] = jnp.where(gidx >= b_i, keyu, _U32(0))
        return carry

    lax.fori_loop(0, SLICES // 4, _mkkeys, 0)

    n0 = b_i
    n1 = N - b_i
    k0f = jnp.maximum(1, n0 >> 3).astype(_F32)
    k1f = jnp.maximum(1, n1 >> 3).astype(_F32)

    def _count2(t0, t1):
        def body(j, carry):
            out = []
            for u in range(8):
                off = (j * 8 + u) * 16
                a0 = carry[2 * u] + jnp.where(
                    key0_v[pl.ds(off, 16)] > t0, _U32(1), _U32(0))
                a1 = carry[2 * u + 1] + jnp.where(
                    key1_v[pl.ds(off, 16)] > t1, _U32(1), _U32(0))
                out += [a0, a1]
            return tuple(out)
        z = jnp.zeros((16,), _U32)
        accs = lax.fori_loop(0, SLICES // 8, body, (z,) * 16)
        a0 = _tree(accs[0::2])
        a1 = _tree(accs[1::2])
        return (jnp.sum(lax.bitcast_convert_type(a0, _I32).astype(_F32)),
                jnp.sum(lax.bitcast_convert_type(a1, _I32).astype(_F32)))

    def _bstep(i, carry):
        lo0, hi0, lo1, hi1 = carry
        par = (i + 1) & 1
        mid0 = lo0 + lax.shift_right_logical(hi0 - lo0, _U32(1))
        mid1 = lo1 + lax.shift_right_logical(hi1 - lo1, _U32(1))
        c0, c1 = _count2(mid0, mid1)
        accv = exchange(par, lanes2(c0, c1))
        c0g = _field(accv, 0)
        c1g = _field(accv, 1)
        up0 = c0g >= k0f
        up1 = c1g >= k1f
        lo0 = jnp.where(up0, mid0 + _U32(1), lo0)
        hi0 = jnp.where(up0, hi0, mid0)
        lo1 = jnp.where(up1, mid1 + _U32(1), lo1)
        hi1 = jnp.where(up1, hi1, mid1)
        return (lo0, hi0, lo1, hi1)

    lo0, hi0, lo1, hi1 = lax.fori_loop(
        0, 30, _bstep,
        (_U32(0x80000000), _U32(0xBF800000),
         _U32(0x80000000), _U32(0xBF800000)))
    t0, t1 = lo0, lo1

    def _csum(j, carry):
        out = []
        for u in range(4):
            off = (j * 4 + u) * 16
            pv = pred_v[pl.ds(off, 16)]
            m0 = key0_v[pl.ds(off, 16)] > t0
            m1 = key1_v[pl.ds(off, 16)] > t1
            a0 = carry[4 * u + 0] + jnp.where(m0, _U32(1), _U32(0))
            a1 = carry[4 * u + 1] + jnp.where(m1, _U32(1), _U32(0))
            s0 = carry[4 * u + 2] + jnp.where(m0, pv, _F32(0.0))
            s1 = carry[4 * u + 3] + jnp.where(m1, pv, _F32(0.0))
            out += [a0, a1, s0, s1]
        return tuple(out)

    zu = jnp.zeros((16,), _U32)
    zf = jnp.zeros((16,), _F32)
    accs = lax.fori_loop(0, SLICES // 4, _csum,
                         (zu, zu, zf, zf) * 4)
    a0 = _tree(accs[0::4])
    a1 = _tree(accs[1::4])
    s0 = _tree(accs[2::4])
    s1 = _tree(accs[3::4])
    cg0 = jnp.sum(lax.bitcast_convert_type(a0, _I32).astype(_F32))
    cg1 = jnp.sum(lax.bitcast_convert_type(a1, _I32).astype(_F32))
    sg0 = jnp.sum(s0)
    sg1 = jnp.sum(s1)

    vals = (jnp.where(it16 == 0, cg0,
            jnp.where(it16 == 1, sg0,
            jnp.where(it16 == 2, cg1,
            jnp.where(it16 == 3, sg1, _F32(0.0))))))
    accf = exchange(1, vals)

    @pl.when(wid == 0)
    def _epilogue():
        C0 = _field(accf, 0)
        S0 = _field(accf, 1)
        C1 = _field(accf, 2)
        S1 = _field(accf, 3)

        tv = jnp.where(it16 == 0, jnp.full((16,), t0), jnp.full((16,), t1))
        topbit = lax.shift_right_logical(tv, _U32(31)) > _U32(0)
        tbits = jnp.where(topbit, tv & _U32(0x7FFFFFFF), ~tv)
        tval = lax.bitcast_convert_type(tbits, _F32)

        kf = jnp.where(it16 == 0, jnp.full((16,), k0f), jnp.full((16,), k1f))
        Cf = jnp.where(it16 == 0, jnp.full((16,), C0), jnp.full((16,), C1))
        Sf = jnp.where(it16 == 0, jnp.full((16,), S0), jnp.full((16,), S1))
        P = (Sf + (kf - Cf) * tval) * _recip(kf)

        ones0 = (n0 - jnp.minimum(n0, c_i)).astype(_F32)
        ones1 = (N - c_i).astype(_F32) - ones0
        nf = jnp.where(it16 == 0, jnp.full((16,), n0.astype(_F32)),
                       jnp.full((16,), n1.astype(_F32)))
        T = (jnp.where(it16 == 0, jnp.full((16,), ones0),
                       jnp.full((16,), ones1))
             * _recip(jnp.maximum(nf, _F32(1.0))))

        lp = jnp.maximum(_safelog(P), _F32(-100.0))
        l1p = jnp.maximum(_safelog(_F32(1.0) - P), _F32(-100.0))
        term = -(T * lp + (_F32(1.0) - T) * l1p)

        maskv = (it16 < 2) & (nf > _F32(0.5))
        nseg = jnp.sum(jnp.where(maskv, _F32(1.0), _F32(0.0)))
        loss = jnp.sum(jnp.where(maskv, term, _F32(0.0))) * _recip(nseg)
        out_v[...] = jnp.full((16,), loss)
        pltpu.sync_copy(out_v, out_hbm)


_mil = functools.partial(
    pl.kernel,
    out_type=jax.ShapeDtypeStruct((16,), _F32),
    mesh=plsc.VectorSubcoreMesh(core_axis_name="c", subcore_axis_name="s",
                                num_cores=1),
    compiler_params=pltpu.CompilerParams(needs_layout_passes=False),
    scratch_types=[
        pltpu.VMEM((CH,), _F32),
        pltpu.VMEM((CH,), _U32),
        pltpu.VMEM((CH,), _U32),
        pltpu.VMEM((CH,), _I32),
        pltpu.VMEM((16,), _F32),
        pltpu.VMEM((NW * 16,), _F32),
        pltpu.VMEM((16,), _F32),
        pltpu.VMEM_SHARED((2, NW * 16), _F32),
    ],
)(_body)


def kernel(pred_dict, label_dict):
    pred = pred_dict[0, :, 0]
    y_row = label_dict[0].astype(_I32)
    seg_key = label_dict[1].astype(_I32)
    out = _mil(pred, seg_key, y_row)
    return out[0]
